# baseline (speedup 1.0000x reference)
import functools
import math

import jax
import jax.numpy as jnp
from jax.experimental import pallas as pl
from jax.experimental.pallas import tpu as pltpu

NUM_HEADS_STATIC = 16


def _fused_body(map_ref, ea_ref, eb_ref, queries_ref, wq_ref, bq_ref, wk_ref,
                wv_ref, bv_ref, wo_ref, bo_ref, out_ref, at_ref, pacc_ref,
                denom_ref, runmax_ref, *, n_heads, head_dim, num_segments,
                hidden, n_blocks, scale):
    i = pl.program_id(0)

    @pl.when(i == 0)
    def _():
        q = jnp.dot(queries_ref[...], wq_ref[...],
                    preferred_element_type=jnp.float32) + bq_ref[...]
        rows = []
        for h in range(n_heads):
            qh = q[:, h * head_dim:(h + 1) * head_dim]
            wkh = wk_ref[:, h * head_dim:(h + 1) * head_dim]
            rows.append(jax.lax.dot_general(
                qh, wkh, (((1,), (1,)), ((), ())),
                preferred_element_type=jnp.float32))
        at_ref[...] = jnp.concatenate(rows, axis=0) * scale
        runmax_ref[...] = jnp.full(runmax_ref.shape, -jnp.inf, jnp.float32)
        pacc_ref[...] = jnp.zeros(pacc_ref.shape, jnp.float32)
        denom_ref[...] = jnp.zeros(denom_ref.shape, jnp.float32)

    ea = ea_ref[...]
    eb = eb_ref[...]
    at = at_ref[...]
    s_a = jax.lax.dot_general(at, ea, (((1,), (1,)), ((), ())),
                              preferred_element_type=jnp.float32)
    s_b = jax.lax.dot_general(at, eb, (((1,), (1,)), ((), ())),
                              preferred_element_type=jnp.float32)
    sh = s_a.shape[0]
    half = s_a.shape[1]
    m = jnp.minimum(map_ref[0], num_segments - 1)
    row_seg = jax.lax.rem(
        jax.lax.broadcasted_iota(jnp.int32, (sh, 1), 0),
        jnp.int32(num_segments))
    mask_a = m[:, :half] == row_seg
    mask_b = m[:, half:] == row_seg

    blkmax = jnp.maximum(
        jnp.max(jnp.where(mask_a, s_a, -jnp.inf), axis=1, keepdims=True),
        jnp.max(jnp.where(mask_b, s_b, -jnp.inf), axis=1, keepdims=True))
    old_max = runmax_ref[...]
    new_max = jnp.maximum(old_max, blkmax)
    alpha = jnp.exp(jnp.where(old_max == -jnp.inf, -jnp.inf,
                              old_max - new_max))
    runmax_ref[...] = new_max
    w_a = jnp.exp(jnp.where(mask_a, s_a - new_max, -jnp.inf))
    w_b = jnp.exp(jnp.where(mask_b, s_b - new_max, -jnp.inf))
    denom_ref[...] = (denom_ref[...] * alpha
                      + jnp.sum(w_a, axis=1, keepdims=True)
                      + jnp.sum(w_b, axis=1, keepdims=True))
    pacc_ref[...] = (pacc_ref[...] * alpha
                     + jnp.dot(w_a, ea, preferred_element_type=jnp.float32)
                     + jnp.dot(w_b, eb, preferred_element_type=jnp.float32))

    @pl.when(i == n_blocks - 1)
    def _():
        denom = denom_ref[...]
        inv = jnp.where(denom > 0.0, 1.0 / denom, 0.0)
        p_mat = pacc_ref[...] * inv
        outs = []
        for h in range(n_heads):
            ph = p_mat[h * num_segments:(h + 1) * num_segments, :]
            wvh = wv_ref[:, h * head_dim:(h + 1) * head_dim]
            outs.append(jnp.dot(ph, wvh, preferred_element_type=jnp.float32))
        per_sample = jnp.concatenate(outs, axis=1)

        ind = (denom > 0.0).astype(jnp.float32)
        zr = jax.lax.broadcasted_iota(jnp.int32, (sh, n_heads), 0)
        zc = jax.lax.broadcasted_iota(jnp.int32, (sh, n_heads), 1)
        z = jnp.where(zr // jnp.int32(num_segments) == zc,
                      jnp.broadcast_to(ind, (sh, n_heads)), 0.0)
        sr = jax.lax.broadcasted_iota(jnp.int32, (num_segments, sh), 0)
        sc = jax.lax.broadcasted_iota(jnp.int32, (num_segments, sh), 1)
        sel = (jax.lax.rem(sc, jnp.int32(num_segments)) == sr).astype(
            jnp.float32)
        ne_sh = jnp.dot(sel, z, preferred_element_type=jnp.float32)
        hr = jax.lax.broadcasted_iota(jnp.int32, (n_heads, hidden), 0)
        hc = jax.lax.broadcasted_iota(jnp.int32, (n_heads, hidden), 1)
        bv_blocks = jnp.where(hc // jnp.int32(head_dim) == hr,
                              bv_ref[...], 0.0)
        per_sample = per_sample + jnp.dot(ne_sh, bv_blocks,
                                          preferred_element_type=jnp.float32)

        out_ref[...] = jnp.dot(per_sample, wo_ref[...],
                               preferred_element_type=jnp.float32) + bo_ref[...]


def kernel(element_embeddings, element_to_sample_map, num_samples, queries,
           Wq, bq, Wkv, bkv, Wo, bo):
    n, d = element_embeddings.shape
    s = queries.shape[0]
    hidden = Wq.shape[1]
    n_heads = NUM_HEADS_STATIC
    head_dim = hidden // n_heads
    sh = n_heads * s
    scale = 1.0 / math.sqrt(head_dim)
    bn = 2048
    n_blocks = n // bn

    map3 = element_to_sample_map.astype(jnp.int32).reshape(n_blocks, 1, bn)
    bq2 = bq.reshape(1, hidden)
    bv2 = bkv[hidden:].reshape(1, hidden)
    bo2 = bo.reshape(1, Wo.shape[1])

    out = pl.pallas_call(
        functools.partial(_fused_body, n_heads=n_heads, head_dim=head_dim,
                          num_segments=s, hidden=hidden, n_blocks=n_blocks,
                          scale=scale),
        grid=(n_blocks,),
        in_specs=[
            pl.BlockSpec((1, 1, bn), lambda i: (i, 0, 0)),
            pl.BlockSpec((bn // 2, d), lambda i: (2 * i, 0)),
            pl.BlockSpec((bn // 2, d), lambda i: (2 * i + 1, 0)),
            pl.BlockSpec(queries.shape, lambda i: (0, 0)),
            pl.BlockSpec(Wq.shape, lambda i: (0, 0)),
            pl.BlockSpec((1, hidden), lambda i: (0, 0)),
            pl.BlockSpec((d, hidden), lambda i: (0, 0)),
            pl.BlockSpec((d, hidden), lambda i: (0, 1)),
            pl.BlockSpec((1, hidden), lambda i: (0, 0)),
            pl.BlockSpec(Wo.shape, lambda i: (0, 0)),
            pl.BlockSpec((1, Wo.shape[1]), lambda i: (0, 0)),
        ],
        out_specs=pl.BlockSpec((s, Wo.shape[1]), lambda i: (0, 0)),
        out_shape=jax.ShapeDtypeStruct((s, Wo.shape[1]), jnp.float32),
        scratch_shapes=[
            pltpu.VMEM((sh, d), jnp.float32),
            pltpu.VMEM((sh, d), jnp.float32),
            pltpu.VMEM((sh, 1), jnp.float32),
            pltpu.VMEM((sh, 1), jnp.float32),
        ],
    )(map3, element_embeddings, element_embeddings, queries, Wq, bq2, Wkv,
      Wkv, bv2, Wo, bo2)
    return out


# speedup vs baseline: 1.0718x; 1.0718x over previous
import functools
import math

import jax
import jax.numpy as jnp
from jax.experimental import pallas as pl
from jax.experimental.pallas import tpu as pltpu

NUM_HEADS_STATIC = 16


def _fused_body(map_ref, e_ref, queries_ref, wq_ref, bq_ref, wk_ref, wkv_ref,
                bv_ref, wo_hbm_ref, bo_ref, out_ref, at_ref, pacc_ref,
                denom_ref, runmax_ref, wv_vmem, wo_vmem, sem_v, sem_o, *,
                n_heads, head_dim, num_segments, hidden, n_blocks, scale):
    i = pl.program_id(0)

    wv_copy = pltpu.make_async_copy(
        wkv_ref.at[:, hidden:2 * hidden], wv_vmem, sem_v)
    wo_copy = pltpu.make_async_copy(wo_hbm_ref, wo_vmem, sem_o)

    @pl.when(i == 0)
    def _():
        wv_copy.start()
        wo_copy.start()
        q = jnp.dot(queries_ref[...], wq_ref[...],
                    preferred_element_type=jnp.float32) + bq_ref[...]
        rows = []
        for h in range(n_heads):
            qh = q[:, h * head_dim:(h + 1) * head_dim]
            wkh = wk_ref[:, h * head_dim:(h + 1) * head_dim]
            rows.append(jax.lax.dot_general(
                qh, wkh, (((1,), (1,)), ((), ())),
                preferred_element_type=jnp.float32))
        at_ref[...] = jnp.concatenate(rows, axis=0) * scale
        runmax_ref[...] = jnp.full(runmax_ref.shape, -jnp.inf, jnp.float32)
        pacc_ref[...] = jnp.zeros(pacc_ref.shape, jnp.float32)
        denom_ref[...] = jnp.zeros(denom_ref.shape, jnp.float32)

    e = e_ref[...]
    s_t = jax.lax.dot_general(at_ref[...], e, (((1,), (1,)), ((), ())),
                              preferred_element_type=jnp.float32)
    sh = s_t.shape[0]
    m_row = jnp.minimum(map_ref[0], num_segments - 1)
    row_seg = jax.lax.rem(
        jax.lax.broadcasted_iota(jnp.int32, (sh, 1), 0),
        jnp.int32(num_segments))
    mask = m_row == row_seg

    blkmax = jnp.max(jnp.where(mask, s_t, -jnp.inf), axis=1,
                     keepdims=True)
    old_max = runmax_ref[...]
    new_max = jnp.maximum(old_max, blkmax)
    alpha = jnp.exp(jnp.where(old_max == -jnp.inf, -jnp.inf,
                              old_max - new_max))
    runmax_ref[...] = new_max
    w = jnp.exp(jnp.where(mask, s_t - new_max, -jnp.inf))
    denom_ref[...] = denom_ref[...] * alpha + jnp.sum(w, axis=1, keepdims=True)
    pacc_ref[...] = pacc_ref[...] * alpha + jnp.dot(
        w, e, preferred_element_type=jnp.float32)

    @pl.when(i == n_blocks - 1)
    def _():
        wv_copy.wait()
        wo_copy.wait()
        denom = denom_ref[...]
        inv = jnp.where(denom > 0.0, 1.0 / denom, 0.0)
        p_mat = pacc_ref[...] * inv
        outs = []
        for h in range(n_heads):
            ph = p_mat[h * num_segments:(h + 1) * num_segments, :]
            wvh = wv_vmem[:, h * head_dim:(h + 1) * head_dim]
            outs.append(jnp.dot(ph, wvh, preferred_element_type=jnp.float32))
        per_sample = jnp.concatenate(outs, axis=1)

        ind = (denom > 0.0).astype(jnp.float32)
        zr = jax.lax.broadcasted_iota(jnp.int32, (sh, n_heads), 0)
        zc = jax.lax.broadcasted_iota(jnp.int32, (sh, n_heads), 1)
        z = jnp.where(zr // jnp.int32(num_segments) == zc,
                      jnp.broadcast_to(ind, (sh, n_heads)), 0.0)
        sr = jax.lax.broadcasted_iota(jnp.int32, (num_segments, sh), 0)
        sc = jax.lax.broadcasted_iota(jnp.int32, (num_segments, sh), 1)
        sel = (jax.lax.rem(sc, jnp.int32(num_segments)) == sr).astype(
            jnp.float32)
        ne_sh = jnp.dot(sel, z, preferred_element_type=jnp.float32)
        hr = jax.lax.broadcasted_iota(jnp.int32, (n_heads, hidden), 0)
        hc = jax.lax.broadcasted_iota(jnp.int32, (n_heads, hidden), 1)
        bv_blocks = jnp.where(hc // jnp.int32(head_dim) == hr,
                              bv_ref[...], 0.0)
        per_sample = per_sample + jnp.dot(ne_sh, bv_blocks,
                                          preferred_element_type=jnp.float32)

        out_ref[...] = jnp.dot(per_sample, wo_vmem[...],
                               preferred_element_type=jnp.float32) + bo_ref[...]


def kernel(element_embeddings, element_to_sample_map, num_samples, queries,
           Wq, bq, Wkv, bkv, Wo, bo):
    n, d = element_embeddings.shape
    s = queries.shape[0]
    hidden = Wq.shape[1]
    n_heads = NUM_HEADS_STATIC
    head_dim = hidden // n_heads
    sh = n_heads * s
    scale = 1.0 / math.sqrt(head_dim)
    bn = 2048
    n_blocks = n // bn

    map3 = element_to_sample_map.astype(jnp.int32).reshape(n_blocks, 1, bn)
    bq2 = bq.reshape(1, hidden)
    bv2 = bkv[hidden:].reshape(1, hidden)
    bo2 = bo.reshape(1, Wo.shape[1])

    out = pl.pallas_call(
        functools.partial(_fused_body, n_heads=n_heads, head_dim=head_dim,
                          num_segments=s, hidden=hidden, n_blocks=n_blocks,
                          scale=scale),
        grid=(n_blocks,),
        in_specs=[
            pl.BlockSpec((1, 1, bn), lambda i: (i, 0, 0)),
            pl.BlockSpec((bn, d), lambda i: (i, 0)),
            pl.BlockSpec(queries.shape, lambda i: (0, 0)),
            pl.BlockSpec(Wq.shape, lambda i: (0, 0)),
            pl.BlockSpec((1, hidden), lambda i: (0, 0)),
            pl.BlockSpec((d, hidden), lambda i: (0, 0)),
            pl.BlockSpec(memory_space=pltpu.MemorySpace.HBM),
            pl.BlockSpec((1, hidden), lambda i: (0, 0)),
            pl.BlockSpec(memory_space=pltpu.MemorySpace.HBM),
            pl.BlockSpec((1, Wo.shape[1]), lambda i: (0, 0)),
        ],
        out_specs=pl.BlockSpec((s, Wo.shape[1]), lambda i: (0, 0)),
        out_shape=jax.ShapeDtypeStruct((s, Wo.shape[1]), jnp.float32),
        scratch_shapes=[
            pltpu.VMEM((sh, d), jnp.float32),
            pltpu.VMEM((sh, d), jnp.float32),
            pltpu.VMEM((sh, 1), jnp.float32),
            pltpu.VMEM((sh, 1), jnp.float32),
            pltpu.VMEM((d, hidden), jnp.float32),
            pltpu.VMEM((d, hidden), jnp.float32),
            pltpu.SemaphoreType.DMA,
            pltpu.SemaphoreType.DMA,
        ],
    )(map3, element_embeddings, queries, Wq, bq2, Wkv, Wkv, bv2, Wo, bo2)
    return out
